# baseline (speedup 1.0000x reference)
import math

import jax
import jax.numpy as jnp
from jax import lax
from jax.experimental import pallas as pl
from jax.experimental.pallas import tpu as pltpu
from jax.experimental.pallas import tpu_sc as plsc

D_MODEL = 768
SCALE = math.sqrt(D_MODEL)
LANES = 16

NUM_CORES = 2
NUM_SUBCORES = 16
NW = NUM_CORES * NUM_SUBCORES

B_TOTAL = 4 * 8192
BPW = B_TOTAL // NW
CHUNK = 32
NCHUNK = BPW // CHUNK
NBUF = 4
PF = 2


def _body(x_hbm, table_hbm, out_hbm, idx_v,
          rows0, rows1, rows2, rows3,
          sg0, sg1, sg2, sg3, so0, so1, so2, so3):
    wid = lax.axis_index("s") * NUM_CORES + lax.axis_index("c")
    base = wid * BPW
    bufs = (rows0, rows1, rows2, rows3)
    gsems = (sg0, sg1, sg2, sg3)
    osems = (so0, so1, so2, so3)

    pltpu.sync_copy(x_hbm.at[wid], idx_v)

    def gather_cp(j, b):
        return pltpu.make_async_copy(table_hbm.at[idx_v.at[j]], bufs[b], gsems[b])

    HALF = CHUNK // 2

    def out_cp(j, b, h):
        src = bufs[b].at[pl.ds(h * HALF, HALF)]
        dst = out_hbm.at[pl.ds(base + j * CHUNK + h * HALF, HALF)]
        return pltpu.make_async_copy(src, dst, osems[b])

    for k in range(PF):
        gather_cp(k, k % NBUF).start()

    @pl.loop(0, NCHUNK, step=NBUF)
    def outer(g):
        for b in range(NBUF):
            j = g + b
            pfb = (b + PF) % NBUF

            @pl.when(j + PF < NCHUNK)
            def _prefetch():
                @pl.when(j + PF - NBUF >= 0)
                def _retire():
                    out_cp(j + PF - NBUF, pfb, 0).wait()
                    out_cp(j + PF - NBUF, pfb, 1).wait()

                gather_cp(j + PF, pfb).start()

            gather_cp(j, b).wait()

            for h in range(2):
                @plsc.parallel_loop(h * HALF, (h + 1) * HALF, step=1, unroll=2)
                def row(r):
                    for d in range(D_MODEL // LANES):
                        sl = pl.ds(d * LANES, LANES)
                        bufs[b][r, sl] = bufs[b][r, sl] * SCALE

                out_cp(j, b, h).start()

    for k in range(NCHUNK - NBUF, NCHUNK):
        out_cp(k, k % NBUF, 0).wait()
        out_cp(k, k % NBUF, 1).wait()


def _make_kernel():
    mesh = plsc.VectorSubcoreMesh(
        core_axis_name="c", subcore_axis_name="s",
        num_cores=NUM_CORES, num_subcores=NUM_SUBCORES,
    )
    return pl.kernel(
        _body,
        out_type=jax.ShapeDtypeStruct((B_TOTAL, D_MODEL), jnp.float32),
        mesh=mesh,
        scratch_types=(
            [pltpu.VMEM((NCHUNK, CHUNK), jnp.int32)]
            + [pltpu.VMEM((CHUNK, D_MODEL), jnp.float32)] * NBUF
            + [pltpu.SemaphoreType.DMA] * (2 * NBUF)
        ),
    )


_lookup = _make_kernel()


def kernel(x, table):
    b, s = x.shape
    x3 = x.astype(jnp.int32).reshape(NW, NCHUNK, CHUNK)
    out = _lookup(x3, table)
    return out.reshape(b, s, D_MODEL)


# speedup vs baseline: 1.0392x; 1.0392x over previous
import math

import jax
import jax.numpy as jnp
from jax import lax
from jax.experimental import pallas as pl
from jax.experimental.pallas import tpu as pltpu
from jax.experimental.pallas import tpu_sc as plsc

D_MODEL = 768
SCALE = math.sqrt(D_MODEL)
LANES = 16

NUM_CORES = 2
NUM_SUBCORES = 16
NW = NUM_CORES * NUM_SUBCORES

B_TOTAL = 4 * 8192
BPW = B_TOTAL // NW
CHUNK = 32
NCHUNK = BPW // CHUNK
NBUF = 4
PF = 2


def _body(x_hbm, table_hbm, out_hbm, idx_v,
          rows0, rows1, rows2, rows3,
          sg0, sg1, sg2, sg3, so0, so1, so2, so3):
    wid = lax.axis_index("s") * NUM_CORES + lax.axis_index("c")
    base = wid * BPW
    bufs = (rows0, rows1, rows2, rows3)
    gsems = (sg0, sg1, sg2, sg3)
    osems = (so0, so1, so2, so3)

    pltpu.sync_copy(x_hbm.at[wid], idx_v)

    def gather_cp(j, b):
        return pltpu.make_async_copy(table_hbm.at[idx_v.at[j]], bufs[b], gsems[b])

    def out_cp(j, b):
        dst = out_hbm.at[pl.ds(base + j * CHUNK, CHUNK)]
        return pltpu.make_async_copy(bufs[b], dst, osems[b])

    for k in range(PF):
        gather_cp(k, k % NBUF).start()

    @pl.loop(0, NCHUNK, step=NBUF)
    def outer(g):
        for b in range(NBUF):
            j = g + b
            pfb = (b + PF) % NBUF

            @pl.when(j + PF < NCHUNK)
            def _prefetch():
                @pl.when(j + PF - NBUF >= 0)
                def _retire():
                    out_cp(j + PF - NBUF, pfb).wait()

                gather_cp(j + PF, pfb).start()

            gather_cp(j, b).wait()

            @plsc.parallel_loop(0, CHUNK, step=1, unroll=2)
            def row(r):
                for d in range(D_MODEL // LANES):
                    sl = pl.ds(d * LANES, LANES)
                    bufs[b][r, sl] = bufs[b][r, sl] * SCALE

            out_cp(j, b).start()

    for k in range(NCHUNK - NBUF, NCHUNK):
        out_cp(k, k % NBUF).wait()


def _make_kernel():
    mesh = plsc.VectorSubcoreMesh(
        core_axis_name="c", subcore_axis_name="s",
        num_cores=NUM_CORES, num_subcores=NUM_SUBCORES,
    )
    return pl.kernel(
        _body,
        out_type=jax.ShapeDtypeStruct((B_TOTAL, D_MODEL), jnp.float32),
        mesh=mesh,
        scratch_types=(
            [pltpu.VMEM((NCHUNK, CHUNK), jnp.int32)]
            + [pltpu.VMEM((CHUNK, D_MODEL), jnp.float32)] * NBUF
            + [pltpu.SemaphoreType.DMA] * (2 * NBUF)
        ),
    )


_lookup = _make_kernel()


def kernel(x, table):
    b, s = x.shape
    x3 = x.astype(jnp.int32).reshape(NW, NCHUNK, CHUNK)
    out = _lookup(x3, table)
    return out.reshape(b, s, D_MODEL)
